# baseline (speedup 1.0000x reference)
import functools

import jax
import jax.numpy as jnp
from jax import lax
from jax.experimental import pallas as pl
from jax.experimental.pallas import tpu as pltpu
from jax.experimental.pallas import tpu_sc as plsc

N = 10000
E = 320000
D_IN = 128
P_DIM = 64
HID = 128
D_OUT = 128
NH = 4
HD = 16

CHUNK = 128
NCH0 = 114
NCH1 = 44
NCHMAX = max(NCH0, NCH1)
C1BASE = 16 * NCH0
EPAD = 16 * (NCH0 + NCH1) * CHUNK
EROWS = EPAD // CHUNK + NCHMAX
NPAD = 10112
STRIPE = NPAD // 16


@functools.cache
def _sc_mesh():
    return plsc.VectorSubcoreMesh(core_axis_name="c", subcore_axis_name="s")


def _scatter_body(srcs_hbm, dsts_hbm, h_hbm, zeros_hbm, out_hbm,
                  sidx, di0, di1, rows0, rows1, acc,
                  is0, is1, gs0, gs1, ss0, ss1):
    cid = lax.axis_index("c")
    sid = lax.axis_index("s")
    base = jnp.where(cid == 0, sid * NCH0, C1BASE + sid * NCH1)
    nch = jnp.where(cid == 0, NCH0, NCH1)
    stripe = pl.ds(sid * STRIPE, STRIPE)
    rbufs = [rows0, rows1]
    dibufs = [di0, di1]
    isems = [is0, is1]
    gsems = [gs0, gs1]
    ssems = [ss0, ss1]

    pltpu.sync_copy(srcs_hbm.at[pl.ds(base, NCHMAX)], sidx)
    pltpu.sync_copy(zeros_hbm.at[stripe], acc.at[stripe])
    plsc.subcore_barrier()

    def start_dstidx(j, b):
        pltpu.async_copy(dsts_hbm.at[base + j], dibufs[b], isems[b])

    def wait_dstidx(b):
        pltpu.make_async_copy(dsts_hbm.at[0], dibufs[b], isems[b]).wait()

    def start_gather(j, b):
        pltpu.async_copy(h_hbm.at[sidx.at[j, 0]], rbufs[b], gsems[b])

    def wait_gather(b):
        pltpu.make_async_copy(h_hbm.at[pl.ds(0, CHUNK)], rbufs[b],
                              gsems[b]).wait()

    def start_scatter(b):
        pltpu.async_copy(rbufs[b], acc.at[dibufs[b].at[0]], ssems[b],
                         add=True)

    def wait_scatter(b):
        pltpu.make_async_copy(rbufs[b], acc.at[pl.ds(0, CHUNK)],
                              ssems[b]).wait()

    for j in (0, 1):
        start_dstidx(j, j)
        start_gather(j, j)
        wait_dstidx(j)
        wait_gather(j)
        start_scatter(j)

    @pl.loop(1, nch // 2)
    def _(t):
        for b in range(2):
            j = 2 * t + b
            wait_scatter(b)
            start_dstidx(j, b)
            start_gather(j, b)
            wait_dstidx(b)
            wait_gather(b)
            start_scatter(b)

    wait_scatter(0)
    wait_scatter(1)
    plsc.subcore_barrier()
    pltpu.sync_copy(acc.at[stripe], out_hbm.at[cid, stripe])


@functools.cache
def _scatter_kernel():
    return pl.kernel(
        _scatter_body,
        out_type=jax.ShapeDtypeStruct((2, NPAD, HID), jnp.float32),
        mesh=_sc_mesh(),
        scratch_types=[
            pltpu.VMEM((NCHMAX, 1, CHUNK), jnp.int32),
            pltpu.VMEM((1, CHUNK), jnp.int32),
            pltpu.VMEM((1, CHUNK), jnp.int32),
            pltpu.VMEM((CHUNK, HID), jnp.float32),
            pltpu.VMEM((CHUNK, HID), jnp.float32),
            pltpu.VMEM_SHARED((NPAD, HID), jnp.float32),
            pltpu.SemaphoreType.DMA,
            pltpu.SemaphoreType.DMA,
            pltpu.SemaphoreType.DMA,
            pltpu.SemaphoreType.DMA,
            pltpu.SemaphoreType.DMA,
            pltpu.SemaphoreType.DMA,
        ],
    )


def _scatter_call(src_p, dst_p, h, zeros128):
    return _scatter_kernel()(src_p, dst_p, h, zeros128)


_RB = 1000
_QB = 400
_KB = 2000


def _qkv_body(x_ref, pew_ref, peb_ref, ipw_ref, ipb_ref, q_ref, k_ref, v_ref):
    pf = lax.dot_general(x_ref[...], pew_ref[...], (((1,), (1,)), ((), ())),
                         preferred_element_type=jnp.float32) + peb_ref[...]
    qkv = lax.dot_general(pf, ipw_ref[...], (((1,), (1,)), ((), ())),
                          preferred_element_type=jnp.float32) + ipb_ref[...]
    q_ref[...] = (qkv[:, :P_DIM] * (1.0 / 4.0)).astype(jnp.bfloat16)
    k_ref[...] = qkv[:, P_DIM:2 * P_DIM].astype(jnp.bfloat16)
    rb = qkv.shape[0]
    v_ref[...] = jnp.concatenate(
        [qkv[:, 2 * P_DIM:],
         jnp.ones((rb, 1), jnp.float32),
         jnp.zeros((rb, P_DIM - 1), jnp.float32)],
        axis=1).astype(jnp.bfloat16)


def _qkv_call(x, pe_W, pe_b, in_proj_w, in_proj_b):
    return pl.pallas_call(
        _qkv_body,
        grid=(N // _RB,),
        in_specs=[
            pl.BlockSpec((_RB, D_IN), lambda i: (i, 0)),
            pl.BlockSpec((P_DIM, D_IN), lambda i: (0, 0)),
            pl.BlockSpec((1, P_DIM), lambda i: (0, 0)),
            pl.BlockSpec((3 * P_DIM, P_DIM), lambda i: (0, 0)),
            pl.BlockSpec((1, 3 * P_DIM), lambda i: (0, 0)),
        ],
        out_specs=[
            pl.BlockSpec((_RB, P_DIM), lambda i: (i, 0)),
            pl.BlockSpec((_RB, P_DIM), lambda i: (i, 0)),
            pl.BlockSpec((_RB, 2 * P_DIM), lambda i: (i, 0)),
        ],
        out_shape=[
            jax.ShapeDtypeStruct((N, P_DIM), jnp.bfloat16),
            jax.ShapeDtypeStruct((N, P_DIM), jnp.bfloat16),
            jax.ShapeDtypeStruct((N, 2 * P_DIM), jnp.bfloat16),
        ],
    )(x, pe_W, pe_b, in_proj_w, in_proj_b)


def _flash_body(q_ref, k_ref, v_ref, o_ref, oacc, dacc):
    j = pl.program_id(1)
    nj = pl.num_programs(1)

    @pl.when(j == 0)
    def _():
        oacc[...] = jnp.zeros_like(oacc)
        dacc[...] = jnp.zeros_like(dacc)

    q = q_ref[...]
    k = k_ref[...]
    v = v_ref[...]
    col = lax.broadcasted_iota(jnp.int32, (1, P_DIM), 1) // HD
    for h in range(NH):
        mask = (col == h)
        kh = jnp.where(mask, k, jnp.bfloat16(0))
        s = lax.dot_general(q, kh, (((1,), (1,)), ((), ())),
                            preferred_element_type=jnp.float32)
        p = jnp.exp(s.astype(jnp.bfloat16))
        o = lax.dot_general(p, v, (((1,), (0,)), ((), ())),
                            preferred_element_type=jnp.float32)
        fmask = mask.astype(jnp.float32)
        oacc[...] += o[:, :P_DIM] * fmask
        dacc[...] += o[:, P_DIM:P_DIM + 1] * fmask

    @pl.when(j == nj - 1)
    def _():
        o_ref[...] = oacc[...] / dacc[...]


def _flash_call(q, k, v):
    return pl.pallas_call(
        _flash_body,
        grid=(N // _QB, N // _KB),
        in_specs=[
            pl.BlockSpec((_QB, P_DIM), lambda i, j: (i, 0)),
            pl.BlockSpec((_KB, P_DIM), lambda i, j: (j, 0)),
            pl.BlockSpec((_KB, 2 * P_DIM), lambda i, j: (j, 0)),
        ],
        out_specs=pl.BlockSpec((_QB, P_DIM), lambda i, j: (i, 0)),
        out_shape=jax.ShapeDtypeStruct((N, P_DIM), jnp.float32),
        scratch_shapes=[
            pltpu.VMEM((_QB, P_DIM), jnp.float32),
            pltpu.VMEM((_QB, P_DIM), jnp.float32),
        ],
    )(q, k, v)


def _l1_body(x_ref, at_ref, w1x_ref, w1po_ref, c1_ref, deg_ref,
             t1s_ref, dinv_ref):
    dinv = lax.rsqrt(1.0 + deg_ref[0, :, 0:1] + deg_ref[1, :, 0:1])
    t1 = lax.dot_general(x_ref[...], w1x_ref[...], (((1,), (1,)), ((), ())),
                         preferred_element_type=jnp.float32)
    t1 += lax.dot_general(at_ref[...], w1po_ref[...], (((1,), (1,)), ((), ())),
                          preferred_element_type=jnp.float32)
    t1 += c1_ref[...]
    t1s_ref[...] = t1 * dinv
    dinv_ref[...] = dinv


def _l1_call(x, attno, W1x, W1po, c1, deg):
    return pl.pallas_call(
        _l1_body,
        grid=(N // _RB,),
        in_specs=[
            pl.BlockSpec((_RB, D_IN), lambda i: (i, 0)),
            pl.BlockSpec((_RB, P_DIM), lambda i: (i, 0)),
            pl.BlockSpec((HID, D_IN), lambda i: (0, 0)),
            pl.BlockSpec((HID, P_DIM), lambda i: (0, 0)),
            pl.BlockSpec((1, HID), lambda i: (0, 0)),
            pl.BlockSpec((2, _RB, HID), lambda i: (0, i, 0)),
        ],
        out_specs=[
            pl.BlockSpec((_RB, HID), lambda i: (i, 0)),
            pl.BlockSpec((_RB, 1), lambda i: (i, 0)),
        ],
        out_shape=[
            jax.ShapeDtypeStruct((N, HID), jnp.float32),
            jax.ShapeDtypeStruct((N, 1), jnp.float32),
        ],
    )(x, attno, W1x, W1po, c1, deg)


def _mid_body(agg_ref, t_ref, dinv_ref, b_ref, w_ref, out_ref):
    u = (agg_ref[0] + agg_ref[1] + t_ref[...]) * dinv_ref[...] + b_ref[...]
    h = jnp.maximum(u, 0.0)
    out_ref[...] = lax.dot_general(h, w_ref[...], (((1,), (1,)), ((), ())),
                                   preferred_element_type=jnp.float32) \
        * dinv_ref[...]


def _mid_call(agg, t, dinv, b, W):
    return pl.pallas_call(
        _mid_body,
        grid=(N // _RB,),
        in_specs=[
            pl.BlockSpec((2, _RB, HID), lambda i: (0, i, 0)),
            pl.BlockSpec((_RB, HID), lambda i: (i, 0)),
            pl.BlockSpec((_RB, 1), lambda i: (i, 0)),
            pl.BlockSpec((1, HID), lambda i: (0, 0)),
            pl.BlockSpec((HID, HID), lambda i: (0, 0)),
        ],
        out_specs=pl.BlockSpec((_RB, HID), lambda i: (i, 0)),
        out_shape=jax.ShapeDtypeStruct((N, HID), jnp.float32),
    )(agg, t, dinv, b, W)


def _final_body(agg_ref, t_ref, dinv_ref, b_ref, out_ref):
    out_ref[...] = (agg_ref[0] + agg_ref[1] + t_ref[...]) * dinv_ref[...] \
        + b_ref[...]


def _final_call(agg, t, dinv, b):
    return pl.pallas_call(
        _final_body,
        grid=(N // _RB,),
        in_specs=[
            pl.BlockSpec((2, _RB, D_OUT), lambda i: (0, i, 0)),
            pl.BlockSpec((_RB, D_OUT), lambda i: (i, 0)),
            pl.BlockSpec((_RB, 1), lambda i: (i, 0)),
            pl.BlockSpec((1, D_OUT), lambda i: (0, 0)),
        ],
        out_specs=pl.BlockSpec((_RB, D_OUT), lambda i: (i, 0)),
        out_shape=jax.ShapeDtypeStruct((N, D_OUT), jnp.float32),
    )(agg, t, dinv, b)


def kernel(x, edge_index, pe_W, pe_b, in_proj_w, in_proj_b,
           out_proj_w, out_proj_b, W1, b1, W2, b2, W3, b3):
    src = edge_index[0].astype(jnp.int32)
    dst = edge_index[1].astype(jnp.int32)
    pad = EROWS * CHUNK - E
    trash = N + jnp.arange(pad, dtype=jnp.int32) % (NPAD - N)
    src_p = jnp.concatenate([src, jnp.zeros((pad,), jnp.int32)])
    dst_p = jnp.concatenate([dst, trash])
    src_p = src_p.reshape(EROWS, 1, CHUNK)
    dst_p = dst_p.reshape(EROWS, 1, CHUNK)
    zeros128 = jnp.zeros((NPAD, HID), jnp.float32)
    ones_n = jnp.ones((N, HID), jnp.float32)

    W1x = W1[:, :D_IN]
    W1p = W1[:, D_IN:]
    W1po = W1p @ out_proj_w
    c1 = (W1p @ out_proj_b).reshape(1, HID)

    deg = _scatter_call(src_p, dst_p, ones_n, zeros128)
    q, k, v = _qkv_call(x, pe_W, pe_b.reshape(1, P_DIM),
                        in_proj_w, in_proj_b.reshape(1, 3 * P_DIM))
    attno = _flash_call(q, k, v)
    t1s, dinv = _l1_call(x, attno, W1x, W1po, c1, deg)
    agg1 = _scatter_call(src_p, dst_p, t1s, zeros128)
    t2s = _mid_call(agg1, t1s, dinv, b1.reshape(1, HID), W2)
    agg2 = _scatter_call(src_p, dst_p, t2s, zeros128)
    t3s = _mid_call(agg2, t2s, dinv, b2.reshape(1, HID), W3)
    agg3 = _scatter_call(src_p, dst_p, t3s, zeros128)
    return _final_call(agg3, t3s, dinv, b3.reshape(1, D_OUT))


# speedup vs baseline: 1.1038x; 1.1038x over previous
import functools

import jax
import jax.numpy as jnp
from jax import lax
from jax.experimental import pallas as pl
from jax.experimental.pallas import tpu as pltpu
from jax.experimental.pallas import tpu_sc as plsc

N = 10000
E = 320000
D_IN = 128
P_DIM = 64
HID = 128
D_OUT = 128
NH = 4
HD = 16

CHUNK = 128
NCH0 = 120
NCH1 = 38
NCHMAX = max(NCH0, NCH1)
C1BASE = 16 * NCH0
EPAD = 16 * (NCH0 + NCH1) * CHUNK
EROWS = EPAD // CHUNK + NCHMAX
NPAD = 10112
STRIPE = NPAD // 16


@functools.cache
def _sc_mesh():
    return plsc.VectorSubcoreMesh(core_axis_name="c", subcore_axis_name="s")


def _scatter_body(srcs_hbm, dsts_hbm, h_hbm, zeros_hbm, out_hbm,
                  sidx, di0, di1, rows0, rows1, acc,
                  is0, is1, gs0, gs1, ss0, ss1):
    cid = lax.axis_index("c")
    sid = lax.axis_index("s")
    base = jnp.where(cid == 0, sid * NCH0, C1BASE + sid * NCH1)
    nch = jnp.where(cid == 0, NCH0, NCH1)
    stripe = pl.ds(sid * STRIPE, STRIPE)
    rbufs = [rows0, rows1]
    dibufs = [di0, di1]
    isems = [is0, is1]
    gsems = [gs0, gs1]
    ssems = [ss0, ss1]

    pltpu.sync_copy(srcs_hbm.at[pl.ds(base, NCHMAX)], sidx)
    pltpu.sync_copy(zeros_hbm.at[stripe], acc.at[stripe])
    plsc.subcore_barrier()

    def start_dstidx(j, b):
        pltpu.async_copy(dsts_hbm.at[base + j], dibufs[b], isems[b])

    def wait_dstidx(b):
        pltpu.make_async_copy(dsts_hbm.at[0], dibufs[b], isems[b]).wait()

    def start_gather(j, b):
        pltpu.async_copy(h_hbm.at[sidx.at[j, 0]], rbufs[b], gsems[b])

    def wait_gather(b):
        pltpu.make_async_copy(h_hbm.at[pl.ds(0, CHUNK)], rbufs[b],
                              gsems[b]).wait()

    def start_scatter(b):
        pltpu.async_copy(rbufs[b], acc.at[dibufs[b].at[0]], ssems[b],
                         add=True)

    def wait_scatter(b):
        pltpu.make_async_copy(rbufs[b], acc.at[pl.ds(0, CHUNK)],
                              ssems[b]).wait()

    for j in (0, 1):
        start_dstidx(j, j)
        start_gather(j, j)
        wait_dstidx(j)
        wait_gather(j)
        start_scatter(j)

    @pl.loop(1, nch // 2)
    def _(t):
        for b in range(2):
            j = 2 * t + b
            wait_scatter(b)
            start_dstidx(j, b)
            start_gather(j, b)
            wait_dstidx(b)
            wait_gather(b)
            start_scatter(b)

    wait_scatter(0)
    wait_scatter(1)
    plsc.subcore_barrier()
    pltpu.sync_copy(acc.at[stripe], out_hbm.at[cid, stripe])


@functools.cache
def _scatter_kernel():
    return pl.kernel(
        _scatter_body,
        out_type=jax.ShapeDtypeStruct((2, NPAD, HID), jnp.float32),
        mesh=_sc_mesh(),
        scratch_types=[
            pltpu.VMEM((NCHMAX, 1, CHUNK), jnp.int32),
            pltpu.VMEM((1, CHUNK), jnp.int32),
            pltpu.VMEM((1, CHUNK), jnp.int32),
            pltpu.VMEM((CHUNK, HID), jnp.float32),
            pltpu.VMEM((CHUNK, HID), jnp.float32),
            pltpu.VMEM_SHARED((NPAD, HID), jnp.float32),
            pltpu.SemaphoreType.DMA,
            pltpu.SemaphoreType.DMA,
            pltpu.SemaphoreType.DMA,
            pltpu.SemaphoreType.DMA,
            pltpu.SemaphoreType.DMA,
            pltpu.SemaphoreType.DMA,
        ],
    )


def _scatter_call(src_p, dst_p, h, zeros128):
    return _scatter_kernel()(src_p, dst_p, h, zeros128)


_RB = 1000
_QB = 400
_KB = 2000


def _qkv_body(x_ref, pew_ref, peb_ref, ipw_ref, ipb_ref, q_ref, k_ref, v_ref):
    pf = lax.dot_general(x_ref[...], pew_ref[...], (((1,), (1,)), ((), ())),
                         preferred_element_type=jnp.float32) + peb_ref[...]
    qkv = lax.dot_general(pf, ipw_ref[...], (((1,), (1,)), ((), ())),
                          preferred_element_type=jnp.float32) + ipb_ref[...]
    q_ref[...] = (qkv[:, :P_DIM] * (1.0 / 4.0)).astype(jnp.bfloat16)
    k_ref[...] = qkv[:, P_DIM:2 * P_DIM].astype(jnp.bfloat16)
    rb = qkv.shape[0]
    v_ref[...] = jnp.concatenate(
        [qkv[:, 2 * P_DIM:],
         jnp.ones((rb, 1), jnp.float32),
         jnp.zeros((rb, P_DIM - 1), jnp.float32)],
        axis=1).astype(jnp.float8_e4m3fn)


def _qkv_call(x, pe_W, pe_b, in_proj_w, in_proj_b):
    return pl.pallas_call(
        _qkv_body,
        grid=(N // _RB,),
        in_specs=[
            pl.BlockSpec((_RB, D_IN), lambda i: (i, 0)),
            pl.BlockSpec((P_DIM, D_IN), lambda i: (0, 0)),
            pl.BlockSpec((1, P_DIM), lambda i: (0, 0)),
            pl.BlockSpec((3 * P_DIM, P_DIM), lambda i: (0, 0)),
            pl.BlockSpec((1, 3 * P_DIM), lambda i: (0, 0)),
        ],
        out_specs=[
            pl.BlockSpec((_RB, P_DIM), lambda i: (i, 0)),
            pl.BlockSpec((_RB, P_DIM), lambda i: (i, 0)),
            pl.BlockSpec((_RB, 2 * P_DIM), lambda i: (i, 0)),
        ],
        out_shape=[
            jax.ShapeDtypeStruct((N, P_DIM), jnp.bfloat16),
            jax.ShapeDtypeStruct((N, P_DIM), jnp.bfloat16),
            jax.ShapeDtypeStruct((N, 2 * P_DIM), jnp.float8_e4m3fn),
        ],
    )(x, pe_W, pe_b, in_proj_w, in_proj_b)


def _flash_body(q_ref, k_ref, v_ref, o_ref, oacc, dacc):
    j = pl.program_id(1)
    nj = pl.num_programs(1)

    @pl.when(j == 0)
    def _():
        oacc[...] = jnp.zeros_like(oacc)
        dacc[...] = jnp.zeros_like(dacc)

    q = q_ref[...]
    k = k_ref[...]
    v = v_ref[...]
    col = lax.broadcasted_iota(jnp.int32, (1, P_DIM), 1) // HD
    for h in range(NH):
        mask = (col == h)
        kh = jnp.where(mask, k, jnp.bfloat16(0))
        s = lax.dot_general(q, kh, (((1,), (1,)), ((), ())),
                            preferred_element_type=jnp.float32)
        p = jnp.exp(s).astype(jnp.float8_e4m3fn)
        o = lax.dot_general(p, v, (((1,), (0,)), ((), ())),
                            preferred_element_type=jnp.float32)
        fmask = mask.astype(jnp.float32)
        oacc[...] += o[:, :P_DIM] * fmask
        dacc[...] += o[:, P_DIM:P_DIM + 1] * fmask

    @pl.when(j == nj - 1)
    def _():
        o_ref[...] = oacc[...] / dacc[...]


def _flash_call(q, k, v):
    return pl.pallas_call(
        _flash_body,
        grid=(N // _QB, N // _KB),
        in_specs=[
            pl.BlockSpec((_QB, P_DIM), lambda i, j: (i, 0)),
            pl.BlockSpec((_KB, P_DIM), lambda i, j: (j, 0)),
            pl.BlockSpec((_KB, 2 * P_DIM), lambda i, j: (j, 0)),
        ],
        out_specs=pl.BlockSpec((_QB, P_DIM), lambda i, j: (i, 0)),
        out_shape=jax.ShapeDtypeStruct((N, P_DIM), jnp.float32),
        scratch_shapes=[
            pltpu.VMEM((_QB, P_DIM), jnp.float32),
            pltpu.VMEM((_QB, P_DIM), jnp.float32),
        ],
    )(q, k, v)


def _l1_body(x_ref, at_ref, w1x_ref, w1po_ref, c1_ref, deg_ref,
             t1s_ref, dinv_ref):
    dinv = lax.rsqrt(1.0 + deg_ref[0, :, 0:1] + deg_ref[1, :, 0:1])
    t1 = lax.dot_general(x_ref[...], w1x_ref[...], (((1,), (1,)), ((), ())),
                         preferred_element_type=jnp.float32)
    t1 += lax.dot_general(at_ref[...], w1po_ref[...], (((1,), (1,)), ((), ())),
                          preferred_element_type=jnp.float32)
    t1 += c1_ref[...]
    t1s_ref[...] = t1 * dinv
    dinv_ref[...] = dinv


def _l1_call(x, attno, W1x, W1po, c1, deg):
    return pl.pallas_call(
        _l1_body,
        grid=(N // _RB,),
        in_specs=[
            pl.BlockSpec((_RB, D_IN), lambda i: (i, 0)),
            pl.BlockSpec((_RB, P_DIM), lambda i: (i, 0)),
            pl.BlockSpec((HID, D_IN), lambda i: (0, 0)),
            pl.BlockSpec((HID, P_DIM), lambda i: (0, 0)),
            pl.BlockSpec((1, HID), lambda i: (0, 0)),
            pl.BlockSpec((2, _RB, HID), lambda i: (0, i, 0)),
        ],
        out_specs=[
            pl.BlockSpec((_RB, HID), lambda i: (i, 0)),
            pl.BlockSpec((_RB, 1), lambda i: (i, 0)),
        ],
        out_shape=[
            jax.ShapeDtypeStruct((N, HID), jnp.float32),
            jax.ShapeDtypeStruct((N, 1), jnp.float32),
        ],
    )(x, attno, W1x, W1po, c1, deg)


def _mid_body(agg_ref, t_ref, dinv_ref, b_ref, w_ref, out_ref):
    u = (agg_ref[0] + agg_ref[1] + t_ref[...]) * dinv_ref[...] + b_ref[...]
    h = jnp.maximum(u, 0.0)
    out_ref[...] = lax.dot_general(h, w_ref[...], (((1,), (1,)), ((), ())),
                                   preferred_element_type=jnp.float32) \
        * dinv_ref[...]


def _mid_call(agg, t, dinv, b, W):
    return pl.pallas_call(
        _mid_body,
        grid=(N // _RB,),
        in_specs=[
            pl.BlockSpec((2, _RB, HID), lambda i: (0, i, 0)),
            pl.BlockSpec((_RB, HID), lambda i: (i, 0)),
            pl.BlockSpec((_RB, 1), lambda i: (i, 0)),
            pl.BlockSpec((1, HID), lambda i: (0, 0)),
            pl.BlockSpec((HID, HID), lambda i: (0, 0)),
        ],
        out_specs=pl.BlockSpec((_RB, HID), lambda i: (i, 0)),
        out_shape=jax.ShapeDtypeStruct((N, HID), jnp.float32),
    )(agg, t, dinv, b, W)


def _final_body(agg_ref, t_ref, dinv_ref, b_ref, out_ref):
    out_ref[...] = (agg_ref[0] + agg_ref[1] + t_ref[...]) * dinv_ref[...] \
        + b_ref[...]


def _final_call(agg, t, dinv, b):
    return pl.pallas_call(
        _final_body,
        grid=(N // _RB,),
        in_specs=[
            pl.BlockSpec((2, _RB, D_OUT), lambda i: (0, i, 0)),
            pl.BlockSpec((_RB, D_OUT), lambda i: (i, 0)),
            pl.BlockSpec((_RB, 1), lambda i: (i, 0)),
            pl.BlockSpec((1, D_OUT), lambda i: (0, 0)),
        ],
        out_specs=pl.BlockSpec((_RB, D_OUT), lambda i: (i, 0)),
        out_shape=jax.ShapeDtypeStruct((N, D_OUT), jnp.float32),
    )(agg, t, dinv, b)


def kernel(x, edge_index, pe_W, pe_b, in_proj_w, in_proj_b,
           out_proj_w, out_proj_b, W1, b1, W2, b2, W3, b3):
    src = edge_index[0].astype(jnp.int32)
    dst = edge_index[1].astype(jnp.int32)
    pad = EROWS * CHUNK - E
    trash = N + jnp.arange(pad, dtype=jnp.int32) % (NPAD - N)
    src_p = jnp.concatenate([src, jnp.zeros((pad,), jnp.int32)])
    dst_p = jnp.concatenate([dst, trash])
    src_p = src_p.reshape(EROWS, 1, CHUNK)
    dst_p = dst_p.reshape(EROWS, 1, CHUNK)
    zeros128 = jnp.zeros((NPAD, HID), jnp.float32)
    ones_n = jnp.ones((N, HID), jnp.float32)

    W1x = W1[:, :D_IN]
    W1p = W1[:, D_IN:]
    W1po = W1p @ out_proj_w
    c1 = (W1p @ out_proj_b).reshape(1, HID)

    deg = _scatter_call(src_p, dst_p, ones_n, zeros128)
    q, k, v = _qkv_call(x, pe_W, pe_b.reshape(1, P_DIM),
                        in_proj_w, in_proj_b.reshape(1, 3 * P_DIM))
    attno = _flash_call(q, k, v)
    t1s, dinv = _l1_call(x, attno, W1x, W1po, c1, deg)
    agg1 = _scatter_call(src_p, dst_p, t1s, zeros128)
    t2s = _mid_call(agg1, t1s, dinv, b1.reshape(1, HID), W2)
    agg2 = _scatter_call(src_p, dst_p, t2s, zeros128)
    t3s = _mid_call(agg2, t2s, dinv, b2.reshape(1, HID), W3)
    agg3 = _scatter_call(src_p, dst_p, t3s, zeros128)
    return _final_call(agg3, t3s, dinv, b3.reshape(1, D_OUT))


# speedup vs baseline: 1.1810x; 1.0700x over previous
import functools

import jax
import jax.numpy as jnp
from jax import lax
from jax.experimental import pallas as pl
from jax.experimental.pallas import tpu as pltpu
from jax.experimental.pallas import tpu_sc as plsc

N = 10000
E = 320000
D_IN = 128
P_DIM = 64
HID = 128
D_OUT = 128
NH = 4
HD = 16

CHUNK = 128
NCH0 = 120
NCH1 = 38
NCHMAX = max(NCH0, NCH1)
C1BASE = 16 * NCH0
EPAD = 16 * (NCH0 + NCH1) * CHUNK
EROWS = EPAD // CHUNK + NCHMAX
NPAD = 10112
STRIPE = NPAD // 16


@functools.cache
def _sc_mesh():
    return plsc.VectorSubcoreMesh(core_axis_name="c", subcore_axis_name="s")


def _scatter_body(srcs_hbm, dsts_hbm, h_hbm, zeros_hbm, out_hbm,
                  sidx, di0, di1, rows0, rows1, acc,
                  is0, is1, gs0, gs1, ss0, ss1):
    cid = lax.axis_index("c")
    sid = lax.axis_index("s")
    base = jnp.where(cid == 0, sid * NCH0, C1BASE + sid * NCH1)
    nch = jnp.where(cid == 0, NCH0, NCH1)
    stripe = pl.ds(sid * STRIPE, STRIPE)
    rbufs = [rows0, rows1]
    dibufs = [di0, di1]
    isems = [is0, is1]
    gsems = [gs0, gs1]
    ssems = [ss0, ss1]

    pltpu.sync_copy(srcs_hbm.at[pl.ds(base, NCHMAX)], sidx)
    pltpu.sync_copy(zeros_hbm.at[stripe], acc.at[stripe])
    plsc.subcore_barrier()

    def start_dstidx(j, b):
        pltpu.async_copy(dsts_hbm.at[base + j], dibufs[b], isems[b])

    def wait_dstidx(b):
        pltpu.make_async_copy(dsts_hbm.at[0], dibufs[b], isems[b]).wait()

    def start_gather(j, b):
        pltpu.async_copy(h_hbm.at[sidx.at[j, 0]], rbufs[b], gsems[b])

    def wait_gather(b):
        pltpu.make_async_copy(h_hbm.at[pl.ds(0, CHUNK)], rbufs[b],
                              gsems[b]).wait()

    def start_scatter(b):
        pltpu.async_copy(rbufs[b], acc.at[dibufs[b].at[0]], ssems[b],
                         add=True)

    def wait_scatter(b):
        pltpu.make_async_copy(rbufs[b], acc.at[pl.ds(0, CHUNK)],
                              ssems[b]).wait()

    for j in (0, 1):
        start_dstidx(j, j)
        start_gather(j, j)
        wait_dstidx(j)
        wait_gather(j)
        start_scatter(j)

    @pl.loop(1, nch // 2)
    def _(t):
        for b in range(2):
            j = 2 * t + b
            wait_scatter(b)
            start_dstidx(j, b)
            start_gather(j, b)
            wait_dstidx(b)
            wait_gather(b)
            start_scatter(b)

    wait_scatter(0)
    wait_scatter(1)
    plsc.subcore_barrier()
    pltpu.sync_copy(acc.at[stripe], out_hbm.at[cid, stripe])


@functools.cache
def _scatter_kernel():
    return pl.kernel(
        _scatter_body,
        out_type=jax.ShapeDtypeStruct((2, NPAD, HID), jnp.float32),
        mesh=_sc_mesh(),
        scratch_types=[
            pltpu.VMEM((NCHMAX, 1, CHUNK), jnp.int32),
            pltpu.VMEM((1, CHUNK), jnp.int32),
            pltpu.VMEM((1, CHUNK), jnp.int32),
            pltpu.VMEM((CHUNK, HID), jnp.float32),
            pltpu.VMEM((CHUNK, HID), jnp.float32),
            pltpu.VMEM_SHARED((NPAD, HID), jnp.float32),
            pltpu.SemaphoreType.DMA,
            pltpu.SemaphoreType.DMA,
            pltpu.SemaphoreType.DMA,
            pltpu.SemaphoreType.DMA,
            pltpu.SemaphoreType.DMA,
            pltpu.SemaphoreType.DMA,
        ],
    )


def _scatter_call(src_p, dst_p, h, zeros128):
    return _scatter_kernel()(src_p, dst_p, h, zeros128)


_RB = 1000
_QB = 400
_KB = 2000


def _qkv_body(x_ref, pew_ref, peb_ref, ipw_ref, ipb_ref, q_ref, k_ref, v_ref):
    pf = lax.dot_general(x_ref[...], pew_ref[...], (((1,), (1,)), ((), ())),
                         preferred_element_type=jnp.float32) + peb_ref[...]
    qkv = lax.dot_general(pf, ipw_ref[...], (((1,), (1,)), ((), ())),
                          preferred_element_type=jnp.float32) + ipb_ref[...]
    q_ref[...] = (qkv[:, :P_DIM] * (1.0 / 4.0)).astype(jnp.float8_e4m3fn)
    k_ref[...] = qkv[:, P_DIM:2 * P_DIM].astype(jnp.float8_e4m3fn)
    rb = qkv.shape[0]
    v_ref[...] = jnp.concatenate(
        [qkv[:, 2 * P_DIM:],
         jnp.ones((rb, 1), jnp.float32),
         jnp.zeros((rb, P_DIM - 1), jnp.float32)],
        axis=1).astype(jnp.float8_e4m3fn)


def _qkv_call(x, pe_W, pe_b, in_proj_w, in_proj_b):
    return pl.pallas_call(
        _qkv_body,
        grid=(N // _RB,),
        in_specs=[
            pl.BlockSpec((_RB, D_IN), lambda i: (i, 0)),
            pl.BlockSpec((P_DIM, D_IN), lambda i: (0, 0)),
            pl.BlockSpec((1, P_DIM), lambda i: (0, 0)),
            pl.BlockSpec((3 * P_DIM, P_DIM), lambda i: (0, 0)),
            pl.BlockSpec((1, 3 * P_DIM), lambda i: (0, 0)),
        ],
        out_specs=[
            pl.BlockSpec((_RB, P_DIM), lambda i: (i, 0)),
            pl.BlockSpec((_RB, P_DIM), lambda i: (i, 0)),
            pl.BlockSpec((_RB, 2 * P_DIM), lambda i: (i, 0)),
        ],
        out_shape=[
            jax.ShapeDtypeStruct((N, P_DIM), jnp.float8_e4m3fn),
            jax.ShapeDtypeStruct((N, P_DIM), jnp.float8_e4m3fn),
            jax.ShapeDtypeStruct((N, 2 * P_DIM), jnp.float8_e4m3fn),
        ],
    )(x, pe_W, pe_b, in_proj_w, in_proj_b)


def _flash_body(q_ref, k_ref, v_ref, o_ref, oacc, dacc):
    j = pl.program_id(1)
    nj = pl.num_programs(1)

    @pl.when(j == 0)
    def _():
        oacc[...] = jnp.zeros_like(oacc)
        dacc[...] = jnp.zeros_like(dacc)

    q = q_ref[...]
    k = k_ref[...]
    v = v_ref[...]
    col = lax.broadcasted_iota(jnp.int32, (1, P_DIM), 1) // HD
    for h in range(NH):
        mask = (col == h)
        kh = jnp.where(mask, k, jnp.float8_e4m3fn(0))
        s = lax.dot_general(q, kh, (((1,), (1,)), ((), ())),
                            preferred_element_type=jnp.float32)
        p = jnp.exp(s).astype(jnp.float8_e4m3fn)
        o = lax.dot_general(p, v, (((1,), (0,)), ((), ())),
                            preferred_element_type=jnp.float32)
        fmask = mask.astype(jnp.float32)
        oacc[...] += o[:, :P_DIM] * fmask
        dacc[...] += o[:, P_DIM:P_DIM + 1] * fmask

    @pl.when(j == nj - 1)
    def _():
        o_ref[...] = oacc[...] / dacc[...]


def _flash_call(q, k, v):
    return pl.pallas_call(
        _flash_body,
        grid=(N // _QB, N // _KB),
        in_specs=[
            pl.BlockSpec((_QB, P_DIM), lambda i, j: (i, 0)),
            pl.BlockSpec((_KB, P_DIM), lambda i, j: (j, 0)),
            pl.BlockSpec((_KB, 2 * P_DIM), lambda i, j: (j, 0)),
        ],
        out_specs=pl.BlockSpec((_QB, P_DIM), lambda i, j: (i, 0)),
        out_shape=jax.ShapeDtypeStruct((N, P_DIM), jnp.float32),
        scratch_shapes=[
            pltpu.VMEM((_QB, P_DIM), jnp.float32),
            pltpu.VMEM((_QB, P_DIM), jnp.float32),
        ],
    )(q, k, v)


def _l1_body(x_ref, at_ref, w1x_ref, w1po_ref, c1_ref, deg_ref,
             t1s_ref, dinv_ref):
    dinv = lax.rsqrt(1.0 + deg_ref[0, :, 0:1] + deg_ref[1, :, 0:1])
    t1 = lax.dot_general(x_ref[...], w1x_ref[...], (((1,), (1,)), ((), ())),
                         preferred_element_type=jnp.float32)
    t1 += lax.dot_general(at_ref[...], w1po_ref[...], (((1,), (1,)), ((), ())),
                          preferred_element_type=jnp.float32)
    t1 += c1_ref[...]
    t1s_ref[...] = t1 * dinv
    dinv_ref[...] = dinv


def _l1_call(x, attno, W1x, W1po, c1, deg):
    return pl.pallas_call(
        _l1_body,
        grid=(N // _RB,),
        in_specs=[
            pl.BlockSpec((_RB, D_IN), lambda i: (i, 0)),
            pl.BlockSpec((_RB, P_DIM), lambda i: (i, 0)),
            pl.BlockSpec((HID, D_IN), lambda i: (0, 0)),
            pl.BlockSpec((HID, P_DIM), lambda i: (0, 0)),
            pl.BlockSpec((1, HID), lambda i: (0, 0)),
            pl.BlockSpec((2, _RB, HID), lambda i: (0, i, 0)),
        ],
        out_specs=[
            pl.BlockSpec((_RB, HID), lambda i: (i, 0)),
            pl.BlockSpec((_RB, 1), lambda i: (i, 0)),
        ],
        out_shape=[
            jax.ShapeDtypeStruct((N, HID), jnp.float32),
            jax.ShapeDtypeStruct((N, 1), jnp.float32),
        ],
    )(x, attno, W1x, W1po, c1, deg)


def _mid_body(agg_ref, t_ref, dinv_ref, b_ref, w_ref, out_ref):
    u = (agg_ref[0] + agg_ref[1] + t_ref[...]) * dinv_ref[...] + b_ref[...]
    h = jnp.maximum(u, 0.0)
    out_ref[...] = lax.dot_general(h, w_ref[...], (((1,), (1,)), ((), ())),
                                   preferred_element_type=jnp.float32) \
        * dinv_ref[...]


def _mid_call(agg, t, dinv, b, W):
    return pl.pallas_call(
        _mid_body,
        grid=(N // _RB,),
        in_specs=[
            pl.BlockSpec((2, _RB, HID), lambda i: (0, i, 0)),
            pl.BlockSpec((_RB, HID), lambda i: (i, 0)),
            pl.BlockSpec((_RB, 1), lambda i: (i, 0)),
            pl.BlockSpec((1, HID), lambda i: (0, 0)),
            pl.BlockSpec((HID, HID), lambda i: (0, 0)),
        ],
        out_specs=pl.BlockSpec((_RB, HID), lambda i: (i, 0)),
        out_shape=jax.ShapeDtypeStruct((N, HID), jnp.float32),
    )(agg, t, dinv, b, W)


def _final_body(agg_ref, t_ref, dinv_ref, b_ref, out_ref):
    out_ref[...] = (agg_ref[0] + agg_ref[1] + t_ref[...]) * dinv_ref[...] \
        + b_ref[...]


def _final_call(agg, t, dinv, b):
    return pl.pallas_call(
        _final_body,
        grid=(N // _RB,),
        in_specs=[
            pl.BlockSpec((2, _RB, D_OUT), lambda i: (0, i, 0)),
            pl.BlockSpec((_RB, D_OUT), lambda i: (i, 0)),
            pl.BlockSpec((_RB, 1), lambda i: (i, 0)),
            pl.BlockSpec((1, D_OUT), lambda i: (0, 0)),
        ],
        out_specs=pl.BlockSpec((_RB, D_OUT), lambda i: (i, 0)),
        out_shape=jax.ShapeDtypeStruct((N, D_OUT), jnp.float32),
    )(agg, t, dinv, b)


def kernel(x, edge_index, pe_W, pe_b, in_proj_w, in_proj_b,
           out_proj_w, out_proj_b, W1, b1, W2, b2, W3, b3):
    src = edge_index[0].astype(jnp.int32)
    dst = edge_index[1].astype(jnp.int32)
    pad = EROWS * CHUNK - E
    trash = N + jnp.arange(pad, dtype=jnp.int32) % (NPAD - N)
    src_p = jnp.concatenate([src, jnp.zeros((pad,), jnp.int32)])
    dst_p = jnp.concatenate([dst, trash])
    src_p = src_p.reshape(EROWS, 1, CHUNK)
    dst_p = dst_p.reshape(EROWS, 1, CHUNK)
    zeros128 = jnp.zeros((NPAD, HID), jnp.float32)
    ones_n = jnp.ones((N, HID), jnp.float32)

    W1x = W1[:, :D_IN]
    W1p = W1[:, D_IN:]
    W1po = W1p @ out_proj_w
    c1 = (W1p @ out_proj_b).reshape(1, HID)

    deg = _scatter_call(src_p, dst_p, ones_n, zeros128)
    q, k, v = _qkv_call(x, pe_W, pe_b.reshape(1, P_DIM),
                        in_proj_w, in_proj_b.reshape(1, 3 * P_DIM))
    attno = _flash_call(q, k, v)
    t1s, dinv = _l1_call(x, attno, W1x, W1po, c1, deg)
    agg1 = _scatter_call(src_p, dst_p, t1s, zeros128)
    t2s = _mid_call(agg1, t1s, dinv, b1.reshape(1, HID), W2)
    agg2 = _scatter_call(src_p, dst_p, t2s, zeros128)
    t3s = _mid_call(agg2, t2s, dinv, b2.reshape(1, HID), W3)
    agg3 = _scatter_call(src_p, dst_p, t3s, zeros128)
    return _final_call(agg3, t3s, dinv, b3.reshape(1, D_OUT))


# speedup vs baseline: 1.1937x; 1.0108x over previous
import functools

import jax
import jax.numpy as jnp
from jax import lax
from jax.experimental import pallas as pl
from jax.experimental.pallas import tpu as pltpu
from jax.experimental.pallas import tpu_sc as plsc

N = 10000
E = 320000
D_IN = 128
P_DIM = 64
HID = 128
D_OUT = 128
NH = 4
HD = 16

CHUNK = 128
NCH0 = 121
NCH1 = 37
NCHMAX = max(NCH0, NCH1)
C1BASE = 16 * NCH0
EPAD = 16 * (NCH0 + NCH1) * CHUNK
EROWS = EPAD // CHUNK + NCHMAX
NPAD = 10112
STRIPE = NPAD // 16


@functools.cache
def _sc_mesh():
    return plsc.VectorSubcoreMesh(core_axis_name="c", subcore_axis_name="s")


def _scatter_body(srcs_hbm, dsts_hbm, h_hbm, zeros_hbm, out_hbm,
                  si0, si1, si2, di0, di1, di2, rows0, rows1, rows2, acc,
                  sis0, sis1, sis2, dis0, dis1, dis2,
                  gs0, gs1, gs2, ss0, ss1, ss2):
    cid = lax.axis_index("c")
    sid = lax.axis_index("s")
    base = jnp.where(cid == 0, sid * NCH0, C1BASE + sid * NCH1)
    nch = jnp.where(cid == 0, NCH0, NCH1)
    stripe = pl.ds(sid * STRIPE, STRIPE)
    rbufs = [rows0, rows1, rows2]
    sibufs = [si0, si1, si2]
    dibufs = [di0, di1, di2]
    sisems = [sis0, sis1, sis2]
    disems = [dis0, dis1, dis2]
    gsems = [gs0, gs1, gs2]
    ssems = [ss0, ss1, ss2]

    pltpu.sync_copy(zeros_hbm.at[stripe], acc.at[stripe])
    plsc.subcore_barrier()

    def start_sidx(j, b):
        pltpu.async_copy(srcs_hbm.at[base + j], sibufs[b], sisems[b])

    def wait_sidx(b):
        pltpu.make_async_copy(srcs_hbm.at[0], sibufs[b], sisems[b]).wait()

    def start_didx(j, b):
        pltpu.async_copy(dsts_hbm.at[base + j], dibufs[b], disems[b])

    def wait_didx(b):
        pltpu.make_async_copy(dsts_hbm.at[0], dibufs[b], disems[b]).wait()

    def start_gather(b):
        pltpu.async_copy(h_hbm.at[sibufs[b].at[0]], rbufs[b], gsems[b])

    def wait_gather(b):
        pltpu.make_async_copy(h_hbm.at[pl.ds(0, CHUNK)], rbufs[b],
                              gsems[b]).wait()

    def start_scatter(b):
        pltpu.async_copy(rbufs[b], acc.at[dibufs[b].at[0]], ssems[b],
                         add=True)

    def wait_scatter(b):
        pltpu.make_async_copy(rbufs[b], acc.at[pl.ds(0, CHUNK)],
                              ssems[b]).wait()

    def body(j, b, bn, b2, first, mid, last):
        wait_gather(b)
        wait_didx(b)
        start_scatter(b)
        if not first:
            wait_scatter(bn)
        if not last:
            start_didx(j + 1, bn)
            if mid:
                start_sidx(j + 2, b2)
            wait_sidx(bn)
            start_gather(bn)

    start_sidx(0, 0)
    start_sidx(1, 1)
    start_didx(0, 0)
    wait_sidx(0)
    start_gather(0)
    body(0, 0, 1, 2, True, True, False)
    body(1, 1, 2, 0, True, True, False)

    @pl.loop(0, (nch - 4) // 3)
    def _(t):
        for k in range(3):
            j = 3 * t + 2 + k
            b = (2 + k) % 3
            body(j, b, (b + 1) % 3, (b + 2) % 3, False, True, False)

    body(nch - 2, 2, 0, 1, False, False, False)
    body(nch - 1, 0, 1, 2, False, False, True)
    wait_scatter(2)
    wait_scatter(0)
    plsc.subcore_barrier()
    pltpu.sync_copy(acc.at[stripe], out_hbm.at[cid, stripe])


@functools.cache
def _scatter_kernel():
    return pl.kernel(
        _scatter_body,
        out_type=jax.ShapeDtypeStruct((2, NPAD, HID), jnp.float32),
        mesh=_sc_mesh(),
        scratch_types=(
            [pltpu.VMEM((1, CHUNK), jnp.int32)] * 6
            + [pltpu.VMEM((CHUNK, HID), jnp.float32)] * 3
            + [pltpu.VMEM_SHARED((NPAD, HID), jnp.float32)]
            + [pltpu.SemaphoreType.DMA] * 12
        ),
    )


def _scatter_call(src_p, dst_p, h, zeros128):
    return _scatter_kernel()(src_p, dst_p, h, zeros128)


_RB = 1000
_QB = 400
_KB = 2000


def _qkv_body(x_ref, pew_ref, peb_ref, ipw_ref, ipb_ref, q_ref, k_ref, v_ref):
    pf = lax.dot_general(x_ref[...], pew_ref[...], (((1,), (1,)), ((), ())),
                         preferred_element_type=jnp.float32) + peb_ref[...]
    qkv = lax.dot_general(pf, ipw_ref[...], (((1,), (1,)), ((), ())),
                          preferred_element_type=jnp.float32) + ipb_ref[...]
    q_ref[...] = (qkv[:, :P_DIM] * (1.0 / 4.0)).astype(jnp.float8_e4m3fn)
    k_ref[...] = qkv[:, P_DIM:2 * P_DIM].astype(jnp.float8_e4m3fn)
    rb = qkv.shape[0]
    v_ref[...] = jnp.concatenate(
        [qkv[:, 2 * P_DIM:],
         jnp.ones((rb, 1), jnp.float32),
         jnp.zeros((rb, P_DIM - 1), jnp.float32)],
        axis=1).astype(jnp.float8_e4m3fn)


def _qkv_call(x, pe_W, pe_b, in_proj_w, in_proj_b):
    return pl.pallas_call(
        _qkv_body,
        grid=(N // _RB,),
        in_specs=[
            pl.BlockSpec((_RB, D_IN), lambda i: (i, 0)),
            pl.BlockSpec((P_DIM, D_IN), lambda i: (0, 0)),
            pl.BlockSpec((1, P_DIM), lambda i: (0, 0)),
            pl.BlockSpec((3 * P_DIM, P_DIM), lambda i: (0, 0)),
            pl.BlockSpec((1, 3 * P_DIM), lambda i: (0, 0)),
        ],
        out_specs=[
            pl.BlockSpec((_RB, P_DIM), lambda i: (i, 0)),
            pl.BlockSpec((_RB, P_DIM), lambda i: (i, 0)),
            pl.BlockSpec((_RB, 2 * P_DIM), lambda i: (i, 0)),
        ],
        out_shape=[
            jax.ShapeDtypeStruct((N, P_DIM), jnp.float8_e4m3fn),
            jax.ShapeDtypeStruct((N, P_DIM), jnp.float8_e4m3fn),
            jax.ShapeDtypeStruct((N, 2 * P_DIM), jnp.float8_e4m3fn),
        ],
    )(x, pe_W, pe_b, in_proj_w, in_proj_b)


def _flash_body(q_ref, k_ref, v_ref, o_ref, oacc, dacc):
    j = pl.program_id(1)
    nj = pl.num_programs(1)

    @pl.when(j == 0)
    def _():
        oacc[...] = jnp.zeros_like(oacc)
        dacc[...] = jnp.zeros_like(dacc)

    q = q_ref[...]
    k = k_ref[...]
    v = v_ref[...]
    col = lax.broadcasted_iota(jnp.int32, (1, P_DIM), 1) // HD
    for h in range(NH):
        mask = (col == h)
        kh = jnp.where(mask, k, jnp.float8_e4m3fn(0))
        s = lax.dot_general(q, kh, (((1,), (1,)), ((), ())),
                            preferred_element_type=jnp.float32)
        p = jnp.exp(s).astype(jnp.float8_e4m3fn)
        o = lax.dot_general(p, v, (((1,), (0,)), ((), ())),
                            preferred_element_type=jnp.float32)
        fmask = mask.astype(jnp.float32)
        oacc[...] += o[:, :P_DIM] * fmask
        dacc[...] += o[:, P_DIM:P_DIM + 1] * fmask

    @pl.when(j == nj - 1)
    def _():
        o_ref[...] = oacc[...] / dacc[...]


def _flash_call(q, k, v):
    return pl.pallas_call(
        _flash_body,
        grid=(N // _QB, N // _KB),
        in_specs=[
            pl.BlockSpec((_QB, P_DIM), lambda i, j: (i, 0)),
            pl.BlockSpec((_KB, P_DIM), lambda i, j: (j, 0)),
            pl.BlockSpec((_KB, 2 * P_DIM), lambda i, j: (j, 0)),
        ],
        out_specs=pl.BlockSpec((_QB, P_DIM), lambda i, j: (i, 0)),
        out_shape=jax.ShapeDtypeStruct((N, P_DIM), jnp.float32),
        scratch_shapes=[
            pltpu.VMEM((_QB, P_DIM), jnp.float32),
            pltpu.VMEM((_QB, P_DIM), jnp.float32),
        ],
    )(q, k, v)


def _l1_body(x_ref, at_ref, w1x_ref, w1po_ref, c1_ref, deg_ref,
             t1s_ref, dinv_ref):
    dinv = lax.rsqrt(1.0 + deg_ref[0, :, 0:1] + deg_ref[1, :, 0:1])
    t1 = lax.dot_general(x_ref[...], w1x_ref[...], (((1,), (1,)), ((), ())),
                         preferred_element_type=jnp.float32)
    t1 += lax.dot_general(at_ref[...], w1po_ref[...], (((1,), (1,)), ((), ())),
                          preferred_element_type=jnp.float32)
    t1 += c1_ref[...]
    t1s_ref[...] = t1 * dinv
    dinv_ref[...] = dinv


def _l1_call(x, attno, W1x, W1po, c1, deg):
    return pl.pallas_call(
        _l1_body,
        grid=(N // _RB,),
        in_specs=[
            pl.BlockSpec((_RB, D_IN), lambda i: (i, 0)),
            pl.BlockSpec((_RB, P_DIM), lambda i: (i, 0)),
            pl.BlockSpec((HID, D_IN), lambda i: (0, 0)),
            pl.BlockSpec((HID, P_DIM), lambda i: (0, 0)),
            pl.BlockSpec((1, HID), lambda i: (0, 0)),
            pl.BlockSpec((2, _RB, HID), lambda i: (0, i, 0)),
        ],
        out_specs=[
            pl.BlockSpec((_RB, HID), lambda i: (i, 0)),
            pl.BlockSpec((_RB, 1), lambda i: (i, 0)),
        ],
        out_shape=[
            jax.ShapeDtypeStruct((N, HID), jnp.float32),
            jax.ShapeDtypeStruct((N, 1), jnp.float32),
        ],
    )(x, attno, W1x, W1po, c1, deg)


def _mid_body(agg_ref, t_ref, dinv_ref, b_ref, w_ref, out_ref):
    u = (agg_ref[0] + agg_ref[1] + t_ref[...]) * dinv_ref[...] + b_ref[...]
    h = jnp.maximum(u, 0.0)
    out_ref[...] = lax.dot_general(h, w_ref[...], (((1,), (1,)), ((), ())),
                                   preferred_element_type=jnp.float32) \
        * dinv_ref[...]


def _mid_call(agg, t, dinv, b, W):
    return pl.pallas_call(
        _mid_body,
        grid=(N // _RB,),
        in_specs=[
            pl.BlockSpec((2, _RB, HID), lambda i: (0, i, 0)),
            pl.BlockSpec((_RB, HID), lambda i: (i, 0)),
            pl.BlockSpec((_RB, 1), lambda i: (i, 0)),
            pl.BlockSpec((1, HID), lambda i: (0, 0)),
            pl.BlockSpec((HID, HID), lambda i: (0, 0)),
        ],
        out_specs=pl.BlockSpec((_RB, HID), lambda i: (i, 0)),
        out_shape=jax.ShapeDtypeStruct((N, HID), jnp.float32),
    )(agg, t, dinv, b, W)


def _final_body(agg_ref, t_ref, dinv_ref, b_ref, out_ref):
    out_ref[...] = (agg_ref[0] + agg_ref[1] + t_ref[...]) * dinv_ref[...] \
        + b_ref[...]


def _final_call(agg, t, dinv, b):
    return pl.pallas_call(
        _final_body,
        grid=(N // _RB,),
        in_specs=[
            pl.BlockSpec((2, _RB, D_OUT), lambda i: (0, i, 0)),
            pl.BlockSpec((_RB, D_OUT), lambda i: (i, 0)),
            pl.BlockSpec((_RB, 1), lambda i: (i, 0)),
            pl.BlockSpec((1, D_OUT), lambda i: (0, 0)),
        ],
        out_specs=pl.BlockSpec((_RB, D_OUT), lambda i: (i, 0)),
        out_shape=jax.ShapeDtypeStruct((N, D_OUT), jnp.float32),
    )(agg, t, dinv, b)


def kernel(x, edge_index, pe_W, pe_b, in_proj_w, in_proj_b,
           out_proj_w, out_proj_b, W1, b1, W2, b2, W3, b3):
    src = edge_index[0].astype(jnp.int32)
    dst = edge_index[1].astype(jnp.int32)
    pad = EROWS * CHUNK - E
    trash = N + jnp.arange(pad, dtype=jnp.int32) % (NPAD - N)
    src_p = jnp.concatenate([src, jnp.zeros((pad,), jnp.int32)])
    dst_p = jnp.concatenate([dst, trash])
    src_p = src_p.reshape(EROWS, 1, CHUNK)
    dst_p = dst_p.reshape(EROWS, 1, CHUNK)
    zeros128 = jnp.zeros((NPAD, HID), jnp.float32)
    ones_n = jnp.ones((N, HID), jnp.float32)

    W1x = W1[:, :D_IN]
    W1p = W1[:, D_IN:]
    W1po = W1p @ out_proj_w
    c1 = (W1p @ out_proj_b).reshape(1, HID)

    deg = _scatter_call(src_p, dst_p, ones_n, zeros128)
    q, k, v = _qkv_call(x, pe_W, pe_b.reshape(1, P_DIM),
                        in_proj_w, in_proj_b.reshape(1, 3 * P_DIM))
    attno = _flash_call(q, k, v)
    t1s, dinv = _l1_call(x, attno, W1x, W1po, c1, deg)
    agg1 = _scatter_call(src_p, dst_p, t1s, zeros128)
    t2s = _mid_call(agg1, t1s, dinv, b1.reshape(1, HID), W2)
    agg2 = _scatter_call(src_p, dst_p, t2s, zeros128)
    t3s = _mid_call(agg2, t2s, dinv, b2.reshape(1, HID), W3)
    agg3 = _scatter_call(src_p, dst_p, t3s, zeros128)
    return _final_call(agg3, t3s, dinv, b3.reshape(1, D_OUT))


# speedup vs baseline: 1.2140x; 1.0170x over previous
import functools

import jax
import jax.numpy as jnp
from jax import lax
from jax.experimental import pallas as pl
from jax.experimental.pallas import tpu as pltpu
from jax.experimental.pallas import tpu_sc as plsc

N = 10000
E = 320000
D_IN = 128
P_DIM = 64
HID = 128
D_OUT = 128
NH = 4
HD = 16

CHUNK = 128
NCH0 = 127
NCH1 = 31
NCHMAX = max(NCH0, NCH1)
C1BASE = 16 * NCH0
EPAD = 16 * (NCH0 + NCH1) * CHUNK
EROWS = EPAD // CHUNK + NCHMAX
NPAD = 10112
STRIPE = NPAD // 16


@functools.cache
def _sc_mesh():
    return plsc.VectorSubcoreMesh(core_axis_name="c", subcore_axis_name="s")


def _scatter_body(srcs_hbm, dsts_hbm, h_hbm, zeros_hbm, out_hbm,
                  si0, si1, si2, di0, di1, di2, rows0, rows1, rows2, acc,
                  sis0, sis1, sis2, dis0, dis1, dis2,
                  gs0, gs1, gs2, ss0, ss1, ss2):
    cid = lax.axis_index("c")
    sid = lax.axis_index("s")
    base = jnp.where(cid == 0, sid * NCH0, C1BASE + sid * NCH1)
    nch = jnp.where(cid == 0, NCH0, NCH1)
    stripe = pl.ds(sid * STRIPE, STRIPE)
    rbufs = [rows0, rows1, rows2]
    sibufs = [si0, si1, si2]
    dibufs = [di0, di1, di2]
    sisems = [sis0, sis1, sis2]
    disems = [dis0, dis1, dis2]
    gsems = [gs0, gs1, gs2]
    ssems = [ss0, ss1, ss2]

    pltpu.sync_copy(zeros_hbm.at[stripe], acc.at[stripe])
    plsc.subcore_barrier()

    def start_sidx(j, b):
        pltpu.async_copy(srcs_hbm.at[base + j], sibufs[b], sisems[b])

    def wait_sidx(b):
        pltpu.make_async_copy(srcs_hbm.at[0], sibufs[b], sisems[b]).wait()

    def start_didx(j, b):
        pltpu.async_copy(dsts_hbm.at[base + j], dibufs[b], disems[b])

    def wait_didx(b):
        pltpu.make_async_copy(dsts_hbm.at[0], dibufs[b], disems[b]).wait()

    def start_gather(b):
        pltpu.async_copy(h_hbm.at[sibufs[b].at[0]], rbufs[b], gsems[b])

    def wait_gather(b):
        pltpu.make_async_copy(h_hbm.at[pl.ds(0, CHUNK)], rbufs[b],
                              gsems[b]).wait()

    def start_scatter(b):
        pltpu.async_copy(rbufs[b], acc.at[dibufs[b].at[0]], ssems[b],
                         add=True)

    def wait_scatter(b):
        pltpu.make_async_copy(rbufs[b], acc.at[pl.ds(0, CHUNK)],
                              ssems[b]).wait()

    def body(j, b, bn, b2, first, mid, last):
        wait_gather(b)
        wait_didx(b)
        start_scatter(b)
        if not first:
            wait_scatter(bn)
        if not last:
            start_didx(j + 1, bn)
            if mid:
                start_sidx(j + 2, b2)
            wait_sidx(bn)
            start_gather(bn)

    start_sidx(0, 0)
    start_sidx(1, 1)
    start_didx(0, 0)
    wait_sidx(0)
    start_gather(0)
    body(0, 0, 1, 2, True, True, False)
    body(1, 1, 2, 0, True, True, False)

    @pl.loop(0, (nch - 4) // 3)
    def _(t):
        for k in range(3):
            j = 3 * t + 2 + k
            b = (2 + k) % 3
            body(j, b, (b + 1) % 3, (b + 2) % 3, False, True, False)

    body(nch - 2, 2, 0, 1, False, False, False)
    body(nch - 1, 0, 1, 2, False, False, True)
    wait_scatter(2)
    wait_scatter(0)
    plsc.subcore_barrier()
    pltpu.sync_copy(acc.at[stripe], out_hbm.at[cid, stripe])


@functools.cache
def _scatter_kernel():
    return pl.kernel(
        _scatter_body,
        out_type=jax.ShapeDtypeStruct((2, NPAD, HID), jnp.float32),
        mesh=_sc_mesh(),
        scratch_types=(
            [pltpu.VMEM((1, CHUNK), jnp.int32)] * 6
            + [pltpu.VMEM((CHUNK, HID), jnp.float32)] * 3
            + [pltpu.VMEM_SHARED((NPAD, HID), jnp.float32)]
            + [pltpu.SemaphoreType.DMA] * 12
        ),
    )


def _scatter_call(src_p, dst_p, h, zeros128):
    return _scatter_kernel()(src_p, dst_p, h, zeros128)


_RB = 1000
_QB = 400
_KB = 2000


def _qkv_body(x_ref, pew_ref, peb_ref, ipw_ref, ipb_ref, q_ref, k_ref, v_ref):
    pf = lax.dot_general(x_ref[...], pew_ref[...], (((1,), (1,)), ((), ())),
                         preferred_element_type=jnp.float32) + peb_ref[...]
    qkv = lax.dot_general(pf, ipw_ref[...], (((1,), (1,)), ((), ())),
                          preferred_element_type=jnp.float32) + ipb_ref[...]
    q_ref[...] = (qkv[:, :P_DIM] * (1.0 / 4.0)).astype(jnp.float8_e4m3fn)
    k_ref[...] = qkv[:, P_DIM:2 * P_DIM].astype(jnp.float8_e4m3fn)
    rb = qkv.shape[0]
    v_ref[...] = jnp.concatenate(
        [qkv[:, 2 * P_DIM:],
         jnp.ones((rb, 1), jnp.float32),
         jnp.zeros((rb, P_DIM - 1), jnp.float32)],
        axis=1).astype(jnp.float8_e4m3fn)


def _qkv_call(x, pe_W, pe_b, in_proj_w, in_proj_b):
    return pl.pallas_call(
        _qkv_body,
        grid=(N // _RB,),
        in_specs=[
            pl.BlockSpec((_RB, D_IN), lambda i: (i, 0)),
            pl.BlockSpec((P_DIM, D_IN), lambda i: (0, 0)),
            pl.BlockSpec((1, P_DIM), lambda i: (0, 0)),
            pl.BlockSpec((3 * P_DIM, P_DIM), lambda i: (0, 0)),
            pl.BlockSpec((1, 3 * P_DIM), lambda i: (0, 0)),
        ],
        out_specs=[
            pl.BlockSpec((_RB, P_DIM), lambda i: (i, 0)),
            pl.BlockSpec((_RB, P_DIM), lambda i: (i, 0)),
            pl.BlockSpec((_RB, 2 * P_DIM), lambda i: (i, 0)),
        ],
        out_shape=[
            jax.ShapeDtypeStruct((N, P_DIM), jnp.float8_e4m3fn),
            jax.ShapeDtypeStruct((N, P_DIM), jnp.float8_e4m3fn),
            jax.ShapeDtypeStruct((N, 2 * P_DIM), jnp.float8_e4m3fn),
        ],
    )(x, pe_W, pe_b, in_proj_w, in_proj_b)


def _flash_body(q_ref, k_ref, v_ref, o_ref, oacc, dacc):
    j = pl.program_id(1)
    nj = pl.num_programs(1)

    @pl.when(j == 0)
    def _():
        oacc[...] = jnp.zeros_like(oacc)
        dacc[...] = jnp.zeros_like(dacc)

    q = q_ref[...]
    k = k_ref[...]
    v = v_ref[...]
    col = lax.broadcasted_iota(jnp.int32, (1, P_DIM), 1) // HD
    for h in range(NH):
        mask = (col == h)
        kh = jnp.where(mask, k, jnp.float8_e4m3fn(0))
        s = lax.dot_general(q, kh, (((1,), (1,)), ((), ())),
                            preferred_element_type=jnp.float32)
        p = jnp.exp(s).astype(jnp.float8_e4m3fn)
        o = lax.dot_general(p, v, (((1,), (0,)), ((), ())),
                            preferred_element_type=jnp.float32)
        fmask = mask.astype(jnp.float32)
        oacc[...] += o[:, :P_DIM] * fmask
        dacc[...] += o[:, P_DIM:P_DIM + 1] * fmask

    @pl.when(j == nj - 1)
    def _():
        o_ref[...] = oacc[...] / dacc[...]


def _flash_call(q, k, v):
    return pl.pallas_call(
        _flash_body,
        grid=(N // _QB, N // _KB),
        in_specs=[
            pl.BlockSpec((_QB, P_DIM), lambda i, j: (i, 0)),
            pl.BlockSpec((_KB, P_DIM), lambda i, j: (j, 0)),
            pl.BlockSpec((_KB, 2 * P_DIM), lambda i, j: (j, 0)),
        ],
        out_specs=pl.BlockSpec((_QB, P_DIM), lambda i, j: (i, 0)),
        out_shape=jax.ShapeDtypeStruct((N, P_DIM), jnp.float32),
        scratch_shapes=[
            pltpu.VMEM((_QB, P_DIM), jnp.float32),
            pltpu.VMEM((_QB, P_DIM), jnp.float32),
        ],
    )(q, k, v)


def _l1_body(x_ref, at_ref, w1x_ref, w1po_ref, c1_ref, deg_ref,
             t1s_ref, dinv_ref):
    dinv = lax.rsqrt(1.0 + deg_ref[0, :, 0:1] + deg_ref[1, :, 0:1])
    t1 = lax.dot_general(x_ref[...], w1x_ref[...], (((1,), (1,)), ((), ())),
                         preferred_element_type=jnp.float32)
    t1 += lax.dot_general(at_ref[...], w1po_ref[...], (((1,), (1,)), ((), ())),
                          preferred_element_type=jnp.float32)
    t1 += c1_ref[...]
    t1s_ref[...] = t1 * dinv
    dinv_ref[...] = dinv


def _l1_call(x, attno, W1x, W1po, c1, deg):
    return pl.pallas_call(
        _l1_body,
        grid=(N // _RB,),
        in_specs=[
            pl.BlockSpec((_RB, D_IN), lambda i: (i, 0)),
            pl.BlockSpec((_RB, P_DIM), lambda i: (i, 0)),
            pl.BlockSpec((HID, D_IN), lambda i: (0, 0)),
            pl.BlockSpec((HID, P_DIM), lambda i: (0, 0)),
            pl.BlockSpec((1, HID), lambda i: (0, 0)),
            pl.BlockSpec((2, _RB, HID), lambda i: (0, i, 0)),
        ],
        out_specs=[
            pl.BlockSpec((_RB, HID), lambda i: (i, 0)),
            pl.BlockSpec((_RB, 1), lambda i: (i, 0)),
        ],
        out_shape=[
            jax.ShapeDtypeStruct((N, HID), jnp.float32),
            jax.ShapeDtypeStruct((N, 1), jnp.float32),
        ],
    )(x, attno, W1x, W1po, c1, deg)


def _mid_body(agg_ref, t_ref, dinv_ref, b_ref, w_ref, out_ref):
    u = (agg_ref[0] + agg_ref[1] + t_ref[...]) * dinv_ref[...] + b_ref[...]
    h = jnp.maximum(u, 0.0)
    out_ref[...] = lax.dot_general(h, w_ref[...], (((1,), (1,)), ((), ())),
                                   preferred_element_type=jnp.float32) \
        * dinv_ref[...]


def _mid_call(agg, t, dinv, b, W):
    return pl.pallas_call(
        _mid_body,
        grid=(N // _RB,),
        in_specs=[
            pl.BlockSpec((2, _RB, HID), lambda i: (0, i, 0)),
            pl.BlockSpec((_RB, HID), lambda i: (i, 0)),
            pl.BlockSpec((_RB, 1), lambda i: (i, 0)),
            pl.BlockSpec((1, HID), lambda i: (0, 0)),
            pl.BlockSpec((HID, HID), lambda i: (0, 0)),
        ],
        out_specs=pl.BlockSpec((_RB, HID), lambda i: (i, 0)),
        out_shape=jax.ShapeDtypeStruct((N, HID), jnp.float32),
    )(agg, t, dinv, b, W)


def _final_body(agg_ref, t_ref, dinv_ref, b_ref, out_ref):
    out_ref[...] = (agg_ref[0] + agg_ref[1] + t_ref[...]) * dinv_ref[...] \
        + b_ref[...]


def _final_call(agg, t, dinv, b):
    return pl.pallas_call(
        _final_body,
        grid=(N // _RB,),
        in_specs=[
            pl.BlockSpec((2, _RB, D_OUT), lambda i: (0, i, 0)),
            pl.BlockSpec((_RB, D_OUT), lambda i: (i, 0)),
            pl.BlockSpec((_RB, 1), lambda i: (i, 0)),
            pl.BlockSpec((1, D_OUT), lambda i: (0, 0)),
        ],
        out_specs=pl.BlockSpec((_RB, D_OUT), lambda i: (i, 0)),
        out_shape=jax.ShapeDtypeStruct((N, D_OUT), jnp.float32),
    )(agg, t, dinv, b)


def kernel(x, edge_index, pe_W, pe_b, in_proj_w, in_proj_b,
           out_proj_w, out_proj_b, W1, b1, W2, b2, W3, b3):
    src = edge_index[0].astype(jnp.int32)
    dst = edge_index[1].astype(jnp.int32)
    pad = EROWS * CHUNK - E
    trash = N + jnp.arange(pad, dtype=jnp.int32) % (NPAD - N)
    src_p = jnp.concatenate([src, jnp.zeros((pad,), jnp.int32)])
    dst_p = jnp.concatenate([dst, trash])
    src_p = src_p.reshape(EROWS, 1, CHUNK)
    dst_p = dst_p.reshape(EROWS, 1, CHUNK)
    zeros128 = jnp.zeros((NPAD, HID), jnp.float32)
    ones_n = jnp.ones((N, HID), jnp.float32)

    W1x = W1[:, :D_IN]
    W1p = W1[:, D_IN:]
    W1po = W1p @ out_proj_w
    c1 = (W1p @ out_proj_b).reshape(1, HID)

    deg = _scatter_call(src_p, dst_p, ones_n, zeros128)
    q, k, v = _qkv_call(x, pe_W, pe_b.reshape(1, P_DIM),
                        in_proj_w, in_proj_b.reshape(1, 3 * P_DIM))
    attno = _flash_call(q, k, v)
    t1s, dinv = _l1_call(x, attno, W1x, W1po, c1, deg)
    agg1 = _scatter_call(src_p, dst_p, t1s, zeros128)
    t2s = _mid_call(agg1, t1s, dinv, b1.reshape(1, HID), W2)
    agg2 = _scatter_call(src_p, dst_p, t2s, zeros128)
    t3s = _mid_call(agg2, t2s, dinv, b2.reshape(1, HID), W3)
    agg3 = _scatter_call(src_p, dst_p, t3s, zeros128)
    return _final_call(agg3, t3s, dinv, b3.reshape(1, D_OUT))
